# baseline (speedup 1.0000x reference)
import jax
import jax.numpy as jnp
from jax.experimental import pallas as pl

N = 50000
R = 4


def _copy_body(x_ref, o_ref):
    o_ref[...] = x_ref[...]


def _rgcn_layer(x, src, dst, etype, W, root, b, n_nodes):
    out = x @ root + b
    for r in range(R):
        mask = (etype == r).astype(x.dtype)
        h = x @ W[r]
        msg = h[src] * mask[:, None]
        s = jax.ops.segment_sum(msg, dst, num_segments=n_nodes)
        cnt = jax.ops.segment_sum(mask, dst, num_segments=n_nodes)
        out = out + s / jnp.maximum(cnt, 1.0)[:, None]
    return out


def kernel(x_code, x_size, edge_index, edge_type, emb_size, emb_code,
           W0, root0, b0, W1, root1, b1, W2, root2, b2, W3, root3, b3,
           W_lin, b_lin):
    x = jnp.concatenate([emb_size[x_size], emb_code[x_code]], axis=1)
    src, dst = edge_index[0], edge_index[1]
    layers = [(W0, root0, b0), (W1, root1, b1), (W2, root2, b2), (W3, root3, b3)]
    for (W, root, b) in layers:
        x = _rgcn_layer(x, src, dst, edge_type, W, root, b, N)
        x = jax.nn.relu(x)
    out = x @ W_lin + b_lin
    out = pl.pallas_call(
        _copy_body,
        out_shape=jax.ShapeDtypeStruct(out.shape, out.dtype),
    )(out)
    return out


# speedup vs baseline: 18.9091x; 18.9091x over previous
import dataclasses
import functools

import jax
import jax.numpy as jnp
from jax import lax
from jax.experimental import pallas as pl
from jax.experimental.pallas import tpu as pltpu
from jax.experimental.pallas import tpu_sc as plsc

N = 50000
E = 1600000
R = 4
NC = 2
NS = 16
NW = NC * NS
LANES = 16

EPT = 50176
E_PAD = NW * EPT
EROWS = E_PAD // 128
ROWS_PER_TILE = EPT // 128
CHUNKS = EPT // 1024

NPAD = 50176
NPT = NPAD // NS
DR = NPT // 8
PAD_ET = 15

_MESH = plsc.VectorSubcoreMesh(core_axis_name="c", subcore_axis_name="s")

_CP = pltpu.CompilerParams(needs_layout_passes=False,
                           use_tc_tiling_on_sc=False)


def _f32(shape):
    return jax.ShapeDtypeStruct(shape, jnp.float32)


NXP = 50048
XCHUNKS = NXP // 128


@functools.partial(
    pl.kernel,
    out_type=_f32((NXP, 32)),
    mesh=_MESH,
    scratch_types=[
        pltpu.VMEM((1, 128), jnp.int32),
        pltpu.VMEM((128, 32), jnp.float32),
        pltpu.SemaphoreType.DMA,
    ],
    compiler_params=_CP,
)
def _emb_gather(xc_hbm, table_hbm, out_hbm, idx_v, stage_v, sem):
    sc = lax.axis_index("c")
    tid = lax.axis_index("s")
    wid = sc * NS + tid

    @pl.loop(0, 13)
    def _(k):
        cid = wid + k * NW

        @pl.when(cid < XCHUNKS)
        def _():
            pltpu.sync_copy(xc_hbm.at[pl.ds(cid, 1)], idx_v)
            pltpu.async_copy(table_hbm.at[idx_v.at[0]], stage_v, sem).wait()
            pltpu.sync_copy(stage_v, out_hbm.at[pl.ds(cid * 128, 128)])


@functools.partial(
    pl.kernel,
    out_type=(_f32((NPAD, 16)), _f32((NPAD, 16))),
    mesh=_MESH,
    scratch_types=[
        pltpu.VMEM((8, 128), jnp.int32),
        pltpu.VMEM((8, 128), jnp.int32),
        pltpu.VMEM((1024, 16), jnp.float32),
        pltpu.VMEM((DR, 16), jnp.float32),
        pltpu.VMEM_SHARED((NPAD, 16), jnp.float32),
    ],
    compiler_params=_CP,
)
def _count_kernel(dst_hbm, et_hbm, outA_hbm, outB_hbm,
                  dst_v, et_v, stage_v, zbuf_v, acc_sh):
    sc = lax.axis_index("c")
    tid = lax.axis_index("s")
    wid = sc * NS + tid
    zero = jnp.zeros((16,), jnp.float32)
    ones = jnp.ones((16,), jnp.float32)
    lane = lax.iota(jnp.int32, 16)

    @pl.loop(0, DR)
    def _(i):
        zbuf_v[i, :] = zero

    @pl.loop(0, 1024)
    def _(i):
        stage_v[i, :] = zero

    @pl.loop(0, 8)
    def _(k):
        pltpu.sync_copy(zbuf_v, acc_sh.at[pl.ds(tid * NPT + k * DR, DR), :])

    plsc.subcore_barrier()

    @pl.loop(0, CHUNKS)
    def _(c):
        rowbase = wid * ROWS_PER_TILE + c * 8
        pltpu.sync_copy(dst_hbm.at[pl.ds(rowbase, 8)], dst_v)
        pltpu.sync_copy(et_hbm.at[pl.ds(rowbase, 8)], et_v)
        for j in range(8):
            @pl.loop(0, 8)
            def _(g, j=j):
                o = g * 16
                ev = j * 128 + o + lane
                et = et_v[j, pl.ds(o, 16)]
                plsc.store_scatter(stage_v, [ev, et], ones)
        for j in range(8):
            pltpu.sync_copy(stage_v.at[pl.ds(j * 128, 128)],
                            acc_sh.at[dst_v.at[j]], add=True)
        for j in range(8):
            @pl.loop(0, 8)
            def _(g, j=j):
                o = g * 16
                ev = j * 128 + o + lane
                et = et_v[j, pl.ds(o, 16)]
                plsc.store_scatter(stage_v, [ev, et], zero)

    plsc.subcore_barrier()

    @pl.loop(0, 8)
    def _(k):
        rb = tid * NPT + k * DR

        @pl.when(sc == 0)
        def _():
            pltpu.sync_copy(acc_sh.at[pl.ds(rb, DR), :],
                            outA_hbm.at[pl.ds(rb, DR), :])

        @pl.when(sc == 1)
        def _():
            pltpu.sync_copy(acc_sh.at[pl.ds(rb, DR), :],
                            outB_hbm.at[pl.ds(rb, DR), :])


@functools.partial(
    pl.kernel,
    out_type=_f32((EROWS, 128)),
    mesh=_MESH,
    scratch_types=[
        pltpu.VMEM((8, 128), jnp.int32),
        pltpu.VMEM((8, 128), jnp.int32),
        pltpu.VMEM((1024, 16), jnp.float32),
        pltpu.VMEM((1024, 16), jnp.float32),
        pltpu.VMEM((8, 128), jnp.float32),
        pltpu.SemaphoreType.DMA,
    ],
    compiler_params=_CP,
)
def _inv_kernel(dst_hbm, et_hbm, cA_hbm, cB_hbm, out_hbm,
                dst_v, et_v, rowsA_v, rowsB_v, invout_v, sem):
    sc = lax.axis_index("c")
    tid = lax.axis_index("s")
    wid = sc * NS + tid
    lane = lax.iota(jnp.int32, 16)

    @pl.loop(0, CHUNKS)
    def _(c):
        rowbase = wid * ROWS_PER_TILE + c * 8
        pltpu.sync_copy(dst_hbm.at[pl.ds(rowbase, 8)], dst_v)
        pltpu.sync_copy(et_hbm.at[pl.ds(rowbase, 8)], et_v)
        hs = []
        for j in range(8):
            hs.append(pltpu.async_copy(cA_hbm.at[dst_v.at[j]],
                                       rowsA_v.at[pl.ds(j * 128, 128)], sem))
            hs.append(pltpu.async_copy(cB_hbm.at[dst_v.at[j]],
                                       rowsB_v.at[pl.ds(j * 128, 128)], sem))
        for h in hs:
            h.wait()
        for j in range(8):
            @pl.loop(0, 8)
            def _(g, j=j):
                o = g * 16
                ev = j * 128 + o + lane
                et = et_v[j, pl.ds(o, 16)]
                va = plsc.load_gather(rowsA_v, [ev, et])
                vb = plsc.load_gather(rowsB_v, [ev, et])
                inv = 1.0 / jnp.maximum(va + vb, 1.0)
                inv = jnp.where(et < R, inv, 0.0)
                invout_v[j, pl.ds(o, 16)] = inv
        pltpu.sync_copy(invout_v, out_hbm.at[pl.ds(rowbase, 8)])


def _make_edge_kernel(P, RC, CHUNK):
    D = CHUNK // 128
    NCH = EPT // CHUNK

    @functools.partial(
        pl.kernel,
        out_type=_f32((2, NPAD, P)),
        mesh=_MESH,
        scratch_types=[
            pltpu.VMEM((D, 128), jnp.int32),
            pltpu.VMEM((D, 128), jnp.int32),
            pltpu.VMEM((D, 128), jnp.int32),
            pltpu.VMEM((D, 128), jnp.float32),
            pltpu.VMEM((D, 128), jnp.int32),
            pltpu.VMEM((CHUNK, P), jnp.float32),
            pltpu.VMEM((DR, P), jnp.float32),
            pltpu.VMEM_SHARED((NPAD, P), jnp.float32),
            pltpu.SemaphoreType.DMA,
        ],
        compiler_params=_CP,
    )
    def edge_kernel(src_hbm, dst_hbm, et_hbm, inv_hbm, hcat_hbm, out_hbm,
                    src_v, dst_v, et_v, inv_v, ridx_v, stage_v, zbuf_v,
                    acc_sh, sem):
        sc = lax.axis_index("c")
        tid = lax.axis_index("s")
        wid = sc * NS + tid
        zero = jnp.zeros((16,), jnp.float32)
        lane = lax.iota(jnp.int32, 16)

        @pl.loop(0, DR)
        def _(i):
            for jj in range(P // 16):
                zbuf_v[i, pl.ds(jj * 16, 16)] = zero

        @pl.loop(0, 8)
        def _(k):
            pltpu.sync_copy(zbuf_v,
                            acc_sh.at[pl.ds(tid * NPT + k * DR, DR), :])

        plsc.subcore_barrier()

        @pl.loop(0, NCH)
        def _(c):
            rowbase = wid * ROWS_PER_TILE + c * D
            pltpu.sync_copy(src_hbm.at[pl.ds(rowbase, D)], src_v)
            pltpu.sync_copy(dst_hbm.at[pl.ds(rowbase, D)], dst_v)
            pltpu.sync_copy(et_hbm.at[pl.ds(rowbase, D)], et_v)
            pltpu.sync_copy(inv_hbm.at[pl.ds(rowbase, D)], inv_v)
            for j in range(D):
                @pl.loop(0, 8)
                def _(g, j=j):
                    o = g * 16
                    et = et_v[j, pl.ds(o, 16)]
                    s = src_v[j, pl.ds(o, 16)]
                    ridx_v[j, pl.ds(o, 16)] = jnp.where(et < R, et * N + s, 0)
            hs = []
            for j in range(D):
                hs.append(pltpu.async_copy(hcat_hbm.at[ridx_v.at[j]],
                                           stage_v.at[pl.ds(j * 128, 128)],
                                           sem))
            for h in hs:
                h.wait()
            for j in range(D):
                @pl.loop(0, 8)
                def _(g, j=j):
                    o = g * 16
                    ev = j * 128 + o + lane
                    iv = inv_v[j, pl.ds(o, 16)]
                    for col in range(RC):
                        cv = jnp.full((16,), col, jnp.int32)
                        v = plsc.load_gather(stage_v, [ev, cv])
                        plsc.store_scatter(stage_v, [ev, cv], v * iv)
            for j in range(D):
                pltpu.sync_copy(stage_v.at[pl.ds(j * 128, 128)],
                                acc_sh.at[dst_v.at[j]], add=True)

        plsc.subcore_barrier()

        @pl.loop(0, 8)
        def _(k):
            rb = tid * NPT + k * DR
            pltpu.sync_copy(acc_sh.at[pl.ds(rb, DR), :],
                            out_hbm.at[sc, pl.ds(rb, DR), :])

    return edge_kernel


_edge_k = {32: _make_edge_kernel(32, 24, 256),
            161: _make_edge_kernel(16, 16, 1024),
            162: _make_edge_kernel(16, 8, 1024),
            163: _make_edge_kernel(16, 4, 1024)}


_BLK = 5000
_GRID = N // _BLK


def _m0_body(xs_ref, xcode_ref, embsz_ref, W_ref, root_ref, b_ref,
             hcat_ref, xroot_ref):
    xs = xs_ref[0, 0, :]
    oh = (xs[:, None] == lax.broadcasted_iota(jnp.int32, (1, 16), 1))
    xsz = jnp.dot(oh.astype(jnp.float32), embsz_ref[...],
                  preferred_element_type=jnp.float32)
    x = jnp.concatenate([xsz, xcode_ref[...]], axis=1)
    for r in range(R):
        hcat_ref[r] = jnp.dot(x, W_ref[r], preferred_element_type=jnp.float32)
    xroot_ref[...] = jnp.dot(x, root_ref[...],
                             preferred_element_type=jnp.float32) + b_ref[...]


def _m_body(o_prev, xroot_ref, acc_ref, W_ref, root_ref, b_ref,
            hcat_ref, xroot_ref_out):
    x = jnp.maximum(
        xroot_ref[...] + acc_ref[0, :, :o_prev] + acc_ref[1, :, :o_prev], 0.0)
    for r in range(R):
        hcat_ref[r] = jnp.dot(x, W_ref[r], preferred_element_type=jnp.float32)
    xroot_ref_out[...] = jnp.dot(x, root_ref[...],
                                 preferred_element_type=jnp.float32) + b_ref[...]


def _m4_body(xroot_ref, acc_ref, Wlin_ref, blin_ref, out_ref):
    x = jnp.maximum(
        xroot_ref[...] + acc_ref[0, :, :4] + acc_ref[1, :, :4], 0.0)
    out_ref[...] = jnp.dot(x, Wlin_ref[...],
                           preferred_element_type=jnp.float32) + blin_ref[...]


def _full(shape):
    nd = len(shape)
    return pl.BlockSpec(shape, lambda i: (0,) * nd)


def _run_m0(xs3, xcode, embsz, Wp, root, b2, P, o):
    return pl.pallas_call(
        _m0_body,
        grid=(_GRID,),
        in_specs=[
            pl.BlockSpec((1, 1, _BLK), lambda i: (i, 0, 0)),
            pl.BlockSpec((_BLK, 32), lambda i: (i, 0)),
            _full(embsz.shape),
            _full(Wp.shape),
            _full(root.shape),
            _full(b2.shape),
        ],
        out_specs=[
            pl.BlockSpec((R, _BLK, P), lambda i: (0, i, 0)),
            pl.BlockSpec((_BLK, o), lambda i: (i, 0)),
        ],
        out_shape=[_f32((R, N, P)), _f32((N, o))],
    )(xs3, xcode, embsz, Wp, root, b2)


def _run_m(xroot, acc, Wp, root, b2, o_prev, P_prev, P, o):
    return pl.pallas_call(
        functools.partial(_m_body, o_prev),
        grid=(_GRID,),
        in_specs=[
            pl.BlockSpec((_BLK, o_prev), lambda i: (i, 0)),
            pl.BlockSpec((2, _BLK, P_prev), lambda i: (0, i, 0)),
            _full(Wp.shape),
            _full(root.shape),
            _full(b2.shape),
        ],
        out_specs=[
            pl.BlockSpec((R, _BLK, P), lambda i: (0, i, 0)),
            pl.BlockSpec((_BLK, o), lambda i: (i, 0)),
        ],
        out_shape=[_f32((R, N, P)), _f32((N, o))],
    )(xroot, acc, Wp, root, b2)


def _run_m4(xroot, acc, Wlinp, blin2):
    return pl.pallas_call(
        _m4_body,
        grid=(_GRID,),
        in_specs=[
            pl.BlockSpec((_BLK, 4), lambda i: (i, 0)),
            pl.BlockSpec((2, _BLK, 16), lambda i: (0, i, 0)),
            _full(Wlinp.shape),
            _full(blin2.shape),
        ],
        out_specs=pl.BlockSpec((_BLK, 8), lambda i: (i, 0)),
        out_shape=_f32((N, 8)),
    )(xroot, acc, Wlinp, blin2)


def kernel(x_code, x_size, edge_index, edge_type, emb_size, emb_code,
           W0, root0, b0, W1, root1, b1, W2, root2, b2, W3, root3, b3,
           W_lin, b_lin):
    f32 = jnp.float32
    src = edge_index[0].astype(jnp.int32)
    dst = edge_index[1].astype(jnp.int32)
    et = edge_type.astype(jnp.int32)

    npad = E_PAD - E
    src2 = jnp.concatenate([src, jnp.zeros((npad,), jnp.int32)]).reshape(EROWS, 128)
    dst2 = jnp.concatenate([dst, jnp.zeros((npad,), jnp.int32)]).reshape(EROWS, 128)
    et2 = jnp.concatenate([et, jnp.full((npad,), PAD_ET, jnp.int32)]).reshape(EROWS, 128)

    xcp = jnp.concatenate([x_code.astype(jnp.int32),
                           jnp.zeros((NXP - N,), jnp.int32)]).reshape(XCHUNKS, 128)
    xs3 = x_size.astype(jnp.int32).reshape(_GRID, 1, _BLK)

    W0p = jnp.pad(W0.astype(f32), ((0, 0), (0, 0), (0, 8)))
    W1p = W1.astype(f32)
    W2p = jnp.pad(W2.astype(f32), ((0, 0), (0, 0), (0, 8)))
    W3p = jnp.pad(W3.astype(f32), ((0, 0), (0, 0), (0, 12)))
    Wlinp = jnp.pad(W_lin.astype(f32), ((0, 0), (0, 6)))

    b0_2 = b0.reshape(1, -1)
    b1_2 = b1.reshape(1, -1)
    b2_2 = b2.reshape(1, -1)
    b3_2 = b3.reshape(1, -1)
    blin2 = jnp.pad(b_lin, (0, 6)).reshape(1, 8)

    xcode_full = _emb_gather(xcp, emb_code.astype(f32))
    xcode = xcode_full[:N]
    cA, cB = _count_kernel(dst2, et2)
    inv2 = _inv_kernel(dst2, et2, cA, cB)

    hcat0, xroot0 = _run_m0(xs3, xcode, emb_size.astype(f32), W0p, root0, b0_2,
                            32, 24)
    acc0 = _edge_k[32](src2, dst2, et2, inv2, hcat0.reshape(R * N, 32))[:, :N]
    hcat1, xroot1 = _run_m(xroot0, acc0, W1p, root1, b1_2, 24, 32, 16, 16)
    acc1 = _edge_k[161](src2, dst2, et2, inv2, hcat1.reshape(R * N, 16))[:, :N]
    hcat2, xroot2 = _run_m(xroot1, acc1, W2p, root2, b2_2, 16, 16, 16, 8)
    acc2 = _edge_k[162](src2, dst2, et2, inv2, hcat2.reshape(R * N, 16))[:, :N]
    hcat3, xroot3 = _run_m(xroot2, acc2, W3p, root3, b3_2, 8, 16, 16, 4)
    acc3 = _edge_k[163](src2, dst2, et2, inv2, hcat3.reshape(R * N, 16))[:, :N]
    out8 = _run_m4(xroot3, acc3, Wlinp, blin2)
    return out8[:, :2]


# speedup vs baseline: 35.0838x; 1.8554x over previous
import dataclasses
import functools

import jax
import jax.numpy as jnp
from jax import lax
from jax.experimental import pallas as pl
from jax.experimental.pallas import tpu as pltpu
from jax.experimental.pallas import tpu_sc as plsc

N = 50000
E = 1600000
R = 4
NC = 2
NS = 16
NW = NC * NS
LANES = 16

EPT = 50176
E_PAD = NW * EPT
EROWS = E_PAD // 128
ROWS_PER_TILE = EPT // 128
CHUNKS = EPT // 1024

NPAD = 50176
NPT = NPAD // NS
DR = NPT // 8
PAD_ET = 15

_MESH = plsc.VectorSubcoreMesh(core_axis_name="c", subcore_axis_name="s")

_CP = pltpu.CompilerParams(needs_layout_passes=False,
                           use_tc_tiling_on_sc=False)


def _f32(shape):
    return jax.ShapeDtypeStruct(shape, jnp.float32)


NXP = 50048
XCHUNKS = NXP // 128


@functools.partial(
    pl.kernel,
    out_type=_f32((NXP, 32)),
    mesh=_MESH,
    scratch_types=[
        pltpu.VMEM((1, 128), jnp.int32),
        pltpu.VMEM((128, 32), jnp.float32),
        pltpu.SemaphoreType.DMA,
    ],
    compiler_params=_CP,
)
def _emb_gather(xc_hbm, table_hbm, out_hbm, idx_v, stage_v, sem):
    sc = lax.axis_index("c")
    tid = lax.axis_index("s")
    wid = sc * NS + tid

    @pl.loop(0, 13)
    def _(k):
        cid = wid + k * NW

        @pl.when(cid < XCHUNKS)
        def _():
            pltpu.sync_copy(xc_hbm.at[pl.ds(cid, 1)], idx_v)
            pltpu.async_copy(table_hbm.at[idx_v.at[0]], stage_v, sem).wait()
            pltpu.sync_copy(stage_v, out_hbm.at[pl.ds(cid * 128, 128)])


@functools.partial(
    pl.kernel,
    out_type=(_f32((NPAD, 16)), _f32((NPAD, 16))),
    mesh=_MESH,
    scratch_types=[
        pltpu.VMEM((8, 128), jnp.int32),
        pltpu.VMEM((8, 128), jnp.int32),
        pltpu.VMEM((1024, 16), jnp.float32),
        pltpu.VMEM((DR, 16), jnp.float32),
        pltpu.VMEM_SHARED((NPAD, 16), jnp.float32),
    ],
    compiler_params=_CP,
)
def _count_kernel(dst_hbm, et_hbm, outA_hbm, outB_hbm,
                  dst_v, et_v, stage_v, zbuf_v, acc_sh):
    sc = lax.axis_index("c")
    tid = lax.axis_index("s")
    wid = sc * NS + tid
    zero = jnp.zeros((16,), jnp.float32)
    ones = jnp.ones((16,), jnp.float32)
    lane = lax.iota(jnp.int32, 16)

    @pl.loop(0, DR)
    def _(i):
        zbuf_v[i, :] = zero

    @pl.loop(0, 1024)
    def _(i):
        stage_v[i, :] = zero

    @pl.loop(0, 8)
    def _(k):
        pltpu.sync_copy(zbuf_v, acc_sh.at[pl.ds(tid * NPT + k * DR, DR), :])

    plsc.subcore_barrier()

    @pl.loop(0, CHUNKS)
    def _(c):
        rowbase = wid * ROWS_PER_TILE + c * 8
        pltpu.sync_copy(dst_hbm.at[pl.ds(rowbase, 8)], dst_v)
        pltpu.sync_copy(et_hbm.at[pl.ds(rowbase, 8)], et_v)
        for j in range(8):
            @pl.loop(0, 8)
            def _(g, j=j):
                o = g * 16
                ev = j * 128 + o + lane
                et = et_v[j, pl.ds(o, 16)]
                plsc.store_scatter(stage_v, [ev, et], ones)
        for j in range(8):
            pltpu.sync_copy(stage_v.at[pl.ds(j * 128, 128)],
                            acc_sh.at[dst_v.at[j]], add=True)
        for j in range(8):
            @pl.loop(0, 8)
            def _(g, j=j):
                o = g * 16
                ev = j * 128 + o + lane
                et = et_v[j, pl.ds(o, 16)]
                plsc.store_scatter(stage_v, [ev, et], zero)

    plsc.subcore_barrier()

    @pl.loop(0, 8)
    def _(k):
        rb = tid * NPT + k * DR

        @pl.when(sc == 0)
        def _():
            pltpu.sync_copy(acc_sh.at[pl.ds(rb, DR), :],
                            outA_hbm.at[pl.ds(rb, DR), :])

        @pl.when(sc == 1)
        def _():
            pltpu.sync_copy(acc_sh.at[pl.ds(rb, DR), :],
                            outB_hbm.at[pl.ds(rb, DR), :])


@functools.partial(
    pl.kernel,
    out_type=_f32((EROWS, 128)),
    mesh=_MESH,
    scratch_types=[
        pltpu.VMEM((8, 128), jnp.int32),
        pltpu.VMEM((8, 128), jnp.int32),
        pltpu.VMEM((1024, 16), jnp.float32),
        pltpu.VMEM((1024, 16), jnp.float32),
        pltpu.VMEM((8, 128), jnp.float32),
        pltpu.SemaphoreType.DMA,
    ],
    compiler_params=_CP,
)
def _inv_kernel(dst_hbm, et_hbm, cA_hbm, cB_hbm, out_hbm,
                dst_v, et_v, rowsA_v, rowsB_v, invout_v, sem):
    sc = lax.axis_index("c")
    tid = lax.axis_index("s")
    wid = sc * NS + tid
    lane = lax.iota(jnp.int32, 16)

    @pl.loop(0, CHUNKS)
    def _(c):
        rowbase = wid * ROWS_PER_TILE + c * 8
        pltpu.sync_copy(dst_hbm.at[pl.ds(rowbase, 8)], dst_v)
        pltpu.sync_copy(et_hbm.at[pl.ds(rowbase, 8)], et_v)
        hs = []
        for j in range(8):
            hs.append(pltpu.async_copy(cA_hbm.at[dst_v.at[j]],
                                       rowsA_v.at[pl.ds(j * 128, 128)], sem))
            hs.append(pltpu.async_copy(cB_hbm.at[dst_v.at[j]],
                                       rowsB_v.at[pl.ds(j * 128, 128)], sem))
        for h in hs:
            h.wait()
        for j in range(8):
            @pl.loop(0, 8)
            def _(g, j=j):
                o = g * 16
                ev = j * 128 + o + lane
                et = et_v[j, pl.ds(o, 16)]
                va = plsc.load_gather(rowsA_v, [ev, et])
                vb = plsc.load_gather(rowsB_v, [ev, et])
                inv = 1.0 / jnp.maximum(va + vb, 1.0)
                inv = jnp.where(et < R, inv, 0.0)
                invout_v[j, pl.ds(o, 16)] = inv
        pltpu.sync_copy(invout_v, out_hbm.at[pl.ds(rowbase, 8)])


def _make_edge_kernel(P, RC, CHUNK):
    D = CHUNK // 128
    NCH = EPT // CHUNK
    MAIN = (NCH - 2) // 3
    TAIL0 = 3 * MAIN
    ZR = 112
    NZ = NPT // ZR

    @functools.partial(
        pl.kernel,
        out_type=_f32((2, NPAD, P)),
        mesh=_MESH,
        scratch_types=[
            pltpu.VMEM((3, D, 3, 128), jnp.int32),
            pltpu.VMEM((3, D, 128), jnp.float32),
            pltpu.VMEM((3, D, 128), jnp.int32),
            pltpu.VMEM((3, D, 128), jnp.int32),
            pltpu.VMEM((3, CHUNK, P), jnp.float32),
            pltpu.VMEM((ZR, P), jnp.float32),
            pltpu.VMEM_SHARED((NPAD, P), jnp.float32),
            pltpu.SemaphoreType.DMA((3,)),
            pltpu.SemaphoreType.DMA((3,)),
            pltpu.SemaphoreType.DMA((3,)),
        ],
        compiler_params=_CP,
    )
    def edge_kernel(ed_hbm, inv_hbm, hcat_hbm, out_hbm,
                    ed_v, inv_v, ridx_v, dsti_v, stage_v, zbuf_v, acc_sh,
                    sem_ed, sem_g, sem_s):
        sc = lax.axis_index("c")
        tid = lax.axis_index("s")
        wid = sc * NS + tid
        zero = jnp.zeros((16,), jnp.float32)
        lane = lax.iota(jnp.int32, 16)

        def rowbase(c):
            return wid * ROWS_PER_TILE + c * D

        def fire_edata(b, c):
            pltpu.async_copy(ed_hbm.at[pl.ds(rowbase(c), D)], ed_v.at[b],
                             sem_ed.at[b])
            pltpu.async_copy(inv_hbm.at[pl.ds(rowbase(c), D)], inv_v.at[b],
                             sem_ed.at[b])

        def wait_edata(b, c):
            pltpu.make_async_copy(ed_hbm.at[pl.ds(rowbase(c), D)],
                                  ed_v.at[b], sem_ed.at[b]).wait()
            pltpu.make_async_copy(inv_hbm.at[pl.ds(rowbase(c), D)],
                                  inv_v.at[b], sem_ed.at[b]).wait()

        def compute_idx(b):
            for j in range(D):
                @pl.loop(0, 8)
                def _(g, j=j):
                    o = g * 16
                    et = ed_v[b, j, 2, pl.ds(o, 16)]
                    s = ed_v[b, j, 0, pl.ds(o, 16)]
                    ridx_v[b, j, pl.ds(o, 16)] = jnp.where(et < R, et * N + s, 0)
                    dsti_v[b, j, pl.ds(o, 16)] = ed_v[b, j, 1, pl.ds(o, 16)]

        def fire_gathers(b):
            for j in range(D):
                pltpu.async_copy(hcat_hbm.at[ridx_v.at[b, j]],
                                 stage_v.at[b, pl.ds(j * 128, 128)],
                                 sem_g.at[b])

        def wait_gathers(b):
            for j in range(D):
                pltpu.make_async_copy(hcat_hbm.at[ridx_v.at[b, j]],
                                      stage_v.at[b, pl.ds(j * 128, 128)],
                                      sem_g.at[b]).wait()

        def scale(b):
            bv = jnp.full((16,), b, jnp.int32)
            for j in range(D):
                @pl.loop(0, 8)
                def _(g, j=j):
                    o = g * 16
                    ev = j * 128 + o + lane
                    iv = inv_v[b, j, pl.ds(o, 16)]
                    for col in range(RC):
                        cv = jnp.full((16,), col, jnp.int32)
                        v = plsc.load_gather(stage_v, [bv, ev, cv])
                        plsc.store_scatter(stage_v, [bv, ev, cv], v * iv)

        def fire_scatters(b):
            for j in range(D):
                pltpu.async_copy(stage_v.at[b, pl.ds(j * 128, 128)],
                                 acc_sh.at[dsti_v.at[b, j]],
                                 sem_s.at[b], add=True)

        def wait_scatters(b):
            for j in range(D):
                pltpu.make_async_copy(stage_v.at[b, pl.ds(j * 128, 128)],
                                      acc_sh.at[dsti_v.at[b, j]],
                                      sem_s.at[b]).wait()

        if P >= 16:
            @pl.loop(0, ZR)
            def _(i):
                zbuf_v[i, pl.ds(0, 16)] = zero
                if P > 16:
                    zbuf_v[i, pl.ds(P - 16, 16)] = zero
        else:
            @pl.loop(0, ZR * P // 16)
            def _(k):
                flat = k * 16 + lane
                plsc.store_scatter(zbuf_v,
                                   [lax.shift_right_logical(flat, 3),
                                    lax.bitwise_and(flat, 7)], zero)

        @pl.loop(0, NZ)
        def _(k):
            pltpu.sync_copy(zbuf_v,
                            acc_sh.at[pl.ds(tid * NPT + k * ZR, ZR), :])

        plsc.subcore_barrier()

        def phase(b, c, first, last):
            bm = (b + 2) % 3
            if first is not None and not first:
                wait_scatters(b)
            elif first is None:
                @pl.when(c >= 3)
                def _():
                    wait_scatters(b)
            wait_edata(b, c)
            compute_idx(b)
            if last is None or not last:
                fire_edata((b + 1) % 3, c + 1)
            fire_gathers(b)
            if first is not None and c >= 1:
                wait_gathers(bm)
                scale(bm)
                fire_scatters(bm)
            elif first is None:
                @pl.when(c >= 1)
                def _():
                    wait_gathers(bm)
                    scale(bm)
                    fire_scatters(bm)

        fire_edata(0, 0)

        @pl.loop(0, MAIN)
        def _(k):
            for i in range(3):
                phase(i, k * 3 + i, None, None)

        for c in range(TAIL0, NCH):
            phase(c % 3, c, c >= 3, c + 1 >= NCH)

        bl = (NCH - 1) % 3
        wait_gathers(bl)
        scale(bl)
        fire_scatters(bl)
        for b in range(3):
            wait_scatters(b)

        plsc.subcore_barrier()

        @pl.loop(0, NZ)
        def _(k):
            rb = tid * NPT + k * ZR
            pltpu.sync_copy(acc_sh.at[pl.ds(rb, ZR), :],
                            out_hbm.at[sc, pl.ds(rb, ZR), :])

    return edge_kernel


_edge_k = [_make_edge_kernel(24, 24, 256),
           _make_edge_kernel(16, 16, 512),
           _make_edge_kernel(8, 8, 512),
           _make_edge_kernel(8, 4, 512)]


_BLK = 5000
_GRID = N // _BLK


def _m0_body(xs_ref, xcode_ref, embsz_ref, W_ref, root_ref, b_ref,
             hcat_ref, xroot_ref):
    xs = xs_ref[0, 0, :]
    oh = (xs[:, None] == lax.broadcasted_iota(jnp.int32, (1, 16), 1))
    xsz = jnp.dot(oh.astype(jnp.float32), embsz_ref[...],
                  preferred_element_type=jnp.float32)
    x = jnp.concatenate([xsz, xcode_ref[...]], axis=1)
    for r in range(R):
        hcat_ref[r] = jnp.dot(x, W_ref[r], preferred_element_type=jnp.float32)
    xroot_ref[...] = jnp.dot(x, root_ref[...],
                             preferred_element_type=jnp.float32) + b_ref[...]


def _m_body(o_prev, xroot_ref, acc_ref, W_ref, root_ref, b_ref,
            hcat_ref, xroot_ref_out):
    x = jnp.maximum(
        xroot_ref[...] + acc_ref[0, :, :o_prev] + acc_ref[1, :, :o_prev], 0.0)
    for r in range(R):
        hcat_ref[r] = jnp.dot(x, W_ref[r], preferred_element_type=jnp.float32)
    xroot_ref_out[...] = jnp.dot(x, root_ref[...],
                                 preferred_element_type=jnp.float32) + b_ref[...]


def _m4_body(xroot_ref, acc_ref, Wlin_ref, blin_ref, out_ref):
    x = jnp.maximum(
        xroot_ref[...] + acc_ref[0, :, :4] + acc_ref[1, :, :4], 0.0)
    out_ref[...] = jnp.dot(x, Wlin_ref[...],
                           preferred_element_type=jnp.float32) + blin_ref[...]


def _full(shape):
    nd = len(shape)
    return pl.BlockSpec(shape, lambda i: (0,) * nd)


def _run_m0(xs3, xcode, embsz, Wp, root, b2, P, o):
    return pl.pallas_call(
        _m0_body,
        grid=(_GRID,),
        in_specs=[
            pl.BlockSpec((1, 1, _BLK), lambda i: (i, 0, 0)),
            pl.BlockSpec((_BLK, 32), lambda i: (i, 0)),
            _full(embsz.shape),
            _full(Wp.shape),
            _full(root.shape),
            _full(b2.shape),
        ],
        out_specs=[
            pl.BlockSpec((R, _BLK, P), lambda i: (0, i, 0)),
            pl.BlockSpec((_BLK, o), lambda i: (i, 0)),
        ],
        out_shape=[_f32((R, N, P)), _f32((N, o))],
    )(xs3, xcode, embsz, Wp, root, b2)


def _run_m(xroot, acc, Wp, root, b2, o_prev, P_prev, P, o):
    return pl.pallas_call(
        functools.partial(_m_body, o_prev),
        grid=(_GRID,),
        in_specs=[
            pl.BlockSpec((_BLK, o_prev), lambda i: (i, 0)),
            pl.BlockSpec((2, _BLK, P_prev), lambda i: (0, i, 0)),
            _full(Wp.shape),
            _full(root.shape),
            _full(b2.shape),
        ],
        out_specs=[
            pl.BlockSpec((R, _BLK, P), lambda i: (0, i, 0)),
            pl.BlockSpec((_BLK, o), lambda i: (i, 0)),
        ],
        out_shape=[_f32((R, N, P)), _f32((N, o))],
    )(xroot, acc, Wp, root, b2)


def _run_m4(xroot, acc, Wlinp, blin2):
    return pl.pallas_call(
        _m4_body,
        grid=(_GRID,),
        in_specs=[
            pl.BlockSpec((_BLK, 4), lambda i: (i, 0)),
            pl.BlockSpec((2, _BLK, 8), lambda i: (0, i, 0)),
            _full(Wlinp.shape),
            _full(blin2.shape),
        ],
        out_specs=pl.BlockSpec((_BLK, 8), lambda i: (i, 0)),
        out_shape=_f32((N, 8)),
    )(xroot, acc, Wlinp, blin2)


def kernel(x_code, x_size, edge_index, edge_type, emb_size, emb_code,
           W0, root0, b0, W1, root1, b1, W2, root2, b2, W3, root3, b3,
           W_lin, b_lin):
    f32 = jnp.float32
    src = edge_index[0].astype(jnp.int32)
    dst = edge_index[1].astype(jnp.int32)
    et = edge_type.astype(jnp.int32)

    npad = E_PAD - E
    src2 = jnp.concatenate([src, jnp.zeros((npad,), jnp.int32)]).reshape(EROWS, 128)
    dst2 = jnp.concatenate([dst, jnp.zeros((npad,), jnp.int32)]).reshape(EROWS, 128)
    et2 = jnp.concatenate([et, jnp.full((npad,), PAD_ET, jnp.int32)]).reshape(EROWS, 128)

    xcp = jnp.concatenate([x_code.astype(jnp.int32),
                           jnp.zeros((NXP - N,), jnp.int32)]).reshape(XCHUNKS, 128)
    xs3 = x_size.astype(jnp.int32).reshape(_GRID, 1, _BLK)

    ed3 = jnp.stack([src2, dst2, et2], axis=1)

    W0p = W0.astype(f32)
    W1p = W1.astype(f32)
    W2p = W2.astype(f32)
    W3p = jnp.pad(W3.astype(f32), ((0, 0), (0, 0), (0, 4)))
    Wlinp = jnp.pad(W_lin.astype(f32), ((0, 0), (0, 6)))

    b0_2 = b0.reshape(1, -1)
    b1_2 = b1.reshape(1, -1)
    b2_2 = b2.reshape(1, -1)
    b3_2 = b3.reshape(1, -1)
    blin2 = jnp.pad(b_lin, (0, 6)).reshape(1, 8)

    xcode_full = _emb_gather(xcp, emb_code.astype(f32))
    xcode = xcode_full[:N]
    cA, cB = _count_kernel(dst2, et2)
    inv2 = _inv_kernel(dst2, et2, cA, cB)

    hcat0, xroot0 = _run_m0(xs3, xcode, emb_size.astype(f32), W0p, root0, b0_2,
                            24, 24)
    acc0 = _edge_k[0](ed3, inv2, hcat0.reshape(R * N, 24))[:, :N]
    hcat1, xroot1 = _run_m(xroot0, acc0, W1p, root1, b1_2, 24, 24, 16, 16)
    acc1 = _edge_k[1](ed3, inv2, hcat1.reshape(R * N, 16))[:, :N]
    hcat2, xroot2 = _run_m(xroot1, acc1, W2p, root2, b2_2, 16, 16, 8, 8)
    acc2 = _edge_k[2](ed3, inv2, hcat2.reshape(R * N, 8))[:, :N]
    hcat3, xroot3 = _run_m(xroot2, acc2, W3p, root3, b3_2, 8, 8, 8, 4)
    acc3 = _edge_k[3](ed3, inv2, hcat3.reshape(R * N, 8))[:, :N]
    out8 = _run_m4(xroot3, acc3, Wlinp, blin2)
    return out8[:, :2]
